# Initial kernel scaffold; baseline (speedup 1.0000x reference)
import functools

import jax
import jax.numpy as jnp
from jax import lax
from jax.experimental import pallas as pl
from jax.experimental.pallas import tpu as pltpu
from jax.experimental.pallas import tpu_sc as plsc

N = 10000
E = 320000
D = 128
DW = 144
ALPHA = 0.2

NC = 2
NS = 16
NW = NC * NS
EPW = E // NW
K = 80
CHUNKS = EPW // K
RPT = N // NS


def _matmul_body(x_ref, w_ref, o_ref):
    o_ref[...] = jnp.dot(x_ref[...], w_ref[...],
                         preferred_element_type=jnp.float32)


def _combine_body(p_ref, o_ref):
    p = p_ref[...]
    a = p[0] + p[1]
    num = a[:, :D]
    den = a[:, D:D + 1] + 1e-8
    hp = num / den
    o_ref[...] = jnp.where(hp > 0, hp, jnp.expm1(hp))


def _edge_body(h_hbm, src_hbm, dst_hbm, zero_hbm, out_hbm,
               sidx, didx, arows, brows, srows, acc, sem_a, sem_b):
    cid = lax.axis_index("c")
    sid = lax.axis_index("s")
    wid = sid * NC + cid

    pltpu.sync_copy(zero_hbm.at[pl.ds(sid * RPT, RPT)],
                    acc.at[pl.ds(sid * RPT, RPT)])
    plsc.subcore_barrier()

    @pl.loop(0, CHUNKS)
    def _(g):
        base = wid * EPW + g * K
        pltpu.sync_copy(src_hbm.at[pl.ds(base, K)], sidx)
        pltpu.sync_copy(dst_hbm.at[pl.ds(base, K)], didx)
        cp_a = pltpu.async_copy(h_hbm.at[sidx], arows, sem_a)
        cp_b = pltpu.async_copy(h_hbm.at[didx], brows, sem_b)
        cp_a.wait()
        cp_b.wait()

        @pl.loop(0, K)
        def _(j):
            av = [arows[j, pl.ds(16 * k, 16)] for k in range(D // 16)]
            bv = [brows[j, pl.ds(16 * k, 16)] for k in range(D // 16)]
            dp = av[0] * bv[0]
            for k in range(1, D // 16):
                dp = dp + av[k] * bv[k]
            s = jnp.sum(dp)
            sv = jnp.full((16,), s, dtype=jnp.float32)
            ev = jnp.exp(-jnp.maximum(sv, ALPHA * sv))
            for k in range(D // 16):
                srows[j, pl.ds(16 * k, 16)] = bv[k] * ev
            srows[j, pl.ds(D, 16)] = ev

        pltpu.sync_copy(srows, acc.at[sidx], add=True)

    plsc.subcore_barrier()
    pltpu.sync_copy(acc.at[pl.ds(sid * RPT, RPT)],
                    out_hbm.at[cid, pl.ds(sid * RPT, RPT)])


def kernel(x, edge_index, W):
    src = edge_index[0].astype(jnp.int32)
    dst = edge_index[1].astype(jnp.int32)
    zeros = jnp.zeros((N, DW), dtype=jnp.float32)

    h = pl.pallas_call(
        _matmul_body,
        out_shape=jax.ShapeDtypeStruct((N, D), jnp.float32),
    )(x, W)

    mesh = plsc.VectorSubcoreMesh(core_axis_name="c", subcore_axis_name="s")
    edge_kernel = pl.kernel(
        _edge_body,
        out_type=jax.ShapeDtypeStruct((NC, N, DW), jnp.float32),
        mesh=mesh,
        scratch_types=[
            pltpu.VMEM((K,), jnp.int32),
            pltpu.VMEM((K,), jnp.int32),
            pltpu.VMEM((K, D), jnp.float32),
            pltpu.VMEM((K, D), jnp.float32),
            pltpu.VMEM((K, DW), jnp.float32),
            pltpu.VMEM_SHARED((N, DW), jnp.float32),
            pltpu.SemaphoreType.DMA,
            pltpu.SemaphoreType.DMA,
        ],
    )
    partial = edge_kernel(h, src, dst, zeros)

    out = pl.pallas_call(
        _combine_body,
        out_shape=jax.ShapeDtypeStruct((N, D), jnp.float32),
    )(partial)
    return out


# speedup vs baseline: 89.7743x; 89.7743x over previous
import dataclasses

import numpy as np

import jax
import jax.numpy as jnp
from jax import lax
from jax.experimental import pallas as pl
from jax.experimental.pallas import tpu as pltpu
from jax.experimental.pallas import tpu_sc as plsc

N = 10000
E = 320000
D = 128
ALPHA = 0.2

NC = 2
NS = 16
NW = NC * NS
EPW = E // NW
K = 80
CHUNKS = EPW // K
NPAD = 10240
RPT = NPAD // NS
RV = RPT // 16


def _loop_i32(n, body, unroll=1):
    def sbody(g, _):
        body(g)
        return g + jnp.int32(1), None
    lax.scan(sbody, jnp.int32(0), None, length=n, unroll=unroll)


def _matmul_body(x_ref, w_ref, o_ref):
    o_ref[...] = jnp.dot(x_ref[...], w_ref[...],
                         preferred_element_type=jnp.float32,
                         precision=lax.Precision.HIGHEST)


def _combine_body(p_ref, rs_ref, o_ref):
    p = p_ref[...]
    num = p[0, :N] + p[1, :N]
    r = rs_ref[...]
    rsum = r[0] + r[1]
    den = jnp.reshape(rsum, (NPAD, 1))[:N] + 1e-8
    hp = num / den
    o_ref[...] = jnp.where(hp > 0, hp, jnp.exp(hp) - 1.0)


def _edge_body(h_hbm, src_hbm, dst_hbm, zero_hbm, out_hbm, rs_hbm, rst_hbm,
               sidx, didx, arows, brows, srows, rsum, rtmp, racc,
               acc, sem_a, sem_b):
    cid = lax.convert_element_type(lax.axis_index("c"), jnp.int32)
    sid = lax.convert_element_type(lax.axis_index("s"), jnp.int32)
    wid = sid * jnp.int32(NC) + cid
    zvec = jnp.zeros((16,), jnp.float32)
    lane0 = lax.iota(jnp.int32, 16) == 0

    rbase = sid * jnp.int32(RPT)
    pltpu.sync_copy(zero_hbm.at[pl.ds(rbase, RPT)],
                    acc.at[pl.ds(rbase, RPT)])

    def zero_body(i):
        rsum[pl.ds(i * jnp.int32(16), 16)] = zvec
    _loop_i32(NPAD // 16, zero_body)
    plsc.subcore_barrier()

    def chunk_body(g):
        base = wid * jnp.int32(EPW) + g * jnp.int32(K)
        pltpu.sync_copy(src_hbm.at[pl.ds(base, K)], sidx)
        pltpu.sync_copy(dst_hbm.at[pl.ds(base, K)], didx)
        cp_a = pltpu.async_copy(h_hbm.at[sidx], arows, sem_a)
        cp_b = pltpu.async_copy(h_hbm.at[didx], brows, sem_b)
        cp_a.wait()
        cp_b.wait()

        def edge_body(j):
            av = [arows[j, pl.ds(16 * k, 16)] for k in range(D // 16)]
            bv = [brows[j, pl.ds(16 * k, 16)] for k in range(D // 16)]
            dp = av[0] * bv[0]
            for k in range(1, D // 16):
                dp = dp + av[k] * bv[k]
            s = jnp.sum(dp)
            sv = jnp.full((16,), s, dtype=jnp.float32)
            ev = jnp.exp(-jnp.maximum(sv, ALPHA * sv))
            for k in range(D // 16):
                srows[j, pl.ds(16 * k, 16)] = bv[k] * ev
            srcv = plsc.load_gather(sidx, [jnp.full((16,), j, jnp.int32)])
            plsc.addupdate_scatter(rsum, [srcv], ev, mask=lane0)

        _loop_i32(K, edge_body)
        pltpu.sync_copy(srows, acc.at[sidx], add=True)

    _loop_i32(CHUNKS, chunk_body)

    cslab = cid * jnp.int32(NS * NPAD)
    pltpu.sync_copy(rsum, rst_hbm.at[pl.ds(cslab + sid * jnp.int32(NPAD),
                                           NPAD)])
    plsc.subcore_barrier()

    pltpu.sync_copy(rst_hbm.at[pl.ds(cslab + rbase, RPT)], racc)

    def rs_reduce(t):
        off = cslab + (t + jnp.int32(1)) * jnp.int32(NPAD) + rbase
        pltpu.sync_copy(rst_hbm.at[pl.ds(off, RPT)], rtmp)

        def add_body(i):
            o = pl.ds(i * jnp.int32(16), 16)
            racc[o] = racc[o] + rtmp[o]
        _loop_i32(RV, add_body)
    _loop_i32(NS - 1, rs_reduce)

    obase = cid * jnp.int32(NPAD) + rbase
    pltpu.sync_copy(acc.at[pl.ds(rbase, RPT)],
                    out_hbm.at[pl.ds(obase, RPT)])
    pltpu.sync_copy(racc, rs_hbm.at[pl.ds(obase, RPT)])


def kernel(x, edge_index, W):
    out_dtype = jnp.result_type(x.dtype, W.dtype)
    x = x.astype(jnp.float32)
    W = W.astype(jnp.float32)
    src = edge_index[0].astype(jnp.int32)
    dst = edge_index[1].astype(jnp.int32)
    zeros = jnp.zeros((NPAD, D), dtype=jnp.float32)

    h = pl.pallas_call(
        _matmul_body,
        out_shape=jax.ShapeDtypeStruct((N, D), jnp.float32),
    )(x, W)

    mesh = plsc.VectorSubcoreMesh(core_axis_name="c", subcore_axis_name="s")
    cp = pltpu.CompilerParams()
    if "needs_layout_passes" in pltpu.CompilerParams.__dataclass_fields__:
        cp = dataclasses.replace(cp, needs_layout_passes=False)
    edge_kernel = pl.kernel(
        _edge_body,
        compiler_params=cp,
        out_type=(
            jax.ShapeDtypeStruct((NC * NPAD, D), jnp.float32),
            jax.ShapeDtypeStruct((NC * NPAD,), jnp.float32),
            jax.ShapeDtypeStruct((NC * NS * NPAD,), jnp.float32),
        ),
        mesh=mesh,
        scratch_types=[
            pltpu.VMEM((K,), jnp.int32),
            pltpu.VMEM((K,), jnp.int32),
            pltpu.VMEM((K, D), jnp.float32),
            pltpu.VMEM((K, D), jnp.float32),
            pltpu.VMEM((K, D), jnp.float32),
            pltpu.VMEM((NPAD,), jnp.float32),
            pltpu.VMEM((RPT,), jnp.float32),
            pltpu.VMEM((RPT,), jnp.float32),
            pltpu.VMEM_SHARED((NPAD, D), jnp.float32),
            pltpu.SemaphoreType.DMA,
            pltpu.SemaphoreType.DMA,
        ],
    )
    feat, rs, _ = edge_kernel(h, src, dst, zeros)
    feat = feat.reshape(NC, NPAD, D)
    rs = rs.reshape(NC, NPAD)

    out = pl.pallas_call(
        _combine_body,
        out_shape=jax.ShapeDtypeStruct((N, D), jnp.float32),
    )(feat, rs)
    return out.astype(out_dtype)
